# Initial kernel scaffold; baseline (speedup 1.0000x reference)
import jax
import jax.numpy as jnp
from jax.experimental import pallas as pl


def kernel(input_, edge_index, edge_weight, W):
    raise NotImplementedError("write your pallas kernel here")


# speedup vs baseline: 4.5179x; 4.5179x over previous
import functools

import jax
import jax.numpy as jnp
import numpy as np
from jax import lax
from jax.experimental import pallas as pl
from jax.experimental.pallas import tpu as pltpu
from jax.experimental.pallas import tpu_sc as plsc

N = 10000
E = 320000
D = 128

NC = 2
NS = 16
L = 16

EDGES_PER_TILE = E // (NC * NS)
CHUNK = 80
STEPS = EDGES_PER_TILE // CHUNK
STRIPE = 624

_SPLAT_DNUMS = jax.lax.GatherDimensionNumbers(
    offset_dims=(), collapsed_slice_dims=(0,), start_index_map=(0,))


def _sc_segment_sum(input_, src, dst, w):
  mesh = plsc.VectorSubcoreMesh(core_axis_name="c", subcore_axis_name="s")

  @functools.partial(
      pl.kernel,
      out_type=jax.ShapeDtypeStruct((NC, N, D), jnp.float32),
      mesh=mesh,
      scratch_types=[
          pltpu.VMEM((CHUNK,), jnp.int32),
          pltpu.VMEM((CHUNK,), jnp.int32),
          pltpu.VMEM((CHUNK,), jnp.float32),
          pltpu.VMEM((CHUNK, D), jnp.float32),
          pltpu.VMEM_SHARED((N, D), jnp.float32),
          pltpu.SemaphoreType.DMA,
      ],
  )
  def k(input_hbm, src_hbm, dst_hbm, w_hbm, out_hbm,
        src_v, dst_v, w_v, rows_v, acc_sh, sem):
    cid = lax.axis_index("c")
    sid = lax.axis_index("s")

    zeros16 = jnp.zeros((L,), jnp.float32)

    def zrow(i, _):
      for j in range(D // L):
        rows_v[i, pl.ds(j * L, L)] = zeros16
      return _

    lax.fori_loop(0, CHUNK, zrow, 0, unroll=False)
    base_row = sid * STRIPE

    def zcopy(i, _):
      pltpu.sync_copy(rows_v,
                      acc_sh.at[pl.ds(base_row + i * CHUNK, CHUNK)])
      return _

    @pl.when(sid < NS - 1)
    def _():
      lax.fori_loop(0, 7, zcopy, 0, unroll=False)
      pltpu.sync_copy(rows_v.at[pl.ds(0, STRIPE - 560)],
                      acc_sh.at[pl.ds(base_row + 560, STRIPE - 560)])

    @pl.when(sid == NS - 1)
    def _():
      lax.fori_loop(0, 8, zcopy, 0, unroll=False)

    plsc.subcore_barrier()

    edge_base = (cid * NS + sid) * EDGES_PER_TILE

    def step(t, _):
      b = edge_base + t * CHUNK
      pltpu.sync_copy(src_hbm.at[pl.ds(b, CHUNK)], src_v)
      pltpu.sync_copy(dst_hbm.at[pl.ds(b, CHUNK)], dst_v)
      pltpu.sync_copy(w_hbm.at[pl.ds(b, CHUNK)], w_v)
      pltpu.async_copy(input_hbm.at[src_v], rows_v, sem).wait()

      def scale16(a, _):
        w16 = w_v[pl.ds(a * L, L)]
        lane = lax.iota(jnp.int32, L)
        for b in range(L):
          bidx = ((lane * 0) + b).reshape(L, 1)
          wsplat = lax.gather(
              w16, bidx, _SPLAT_DNUMS, slice_sizes=(1,),
              mode=lax.GatherScatterMode.PROMISE_IN_BOUNDS)
          r = a * L + b
          for j in range(D // L):
            sl = pl.ds(j * L, L)
            rows_v[r, sl] = rows_v[r, sl] * wsplat
        return _

      lax.fori_loop(0, CHUNK // L, scale16, 0, unroll=False)
      pltpu.sync_copy(rows_v, acc_sh.at[dst_v], add=True)
      return _

    lax.fori_loop(0, STEPS, step, 0, unroll=False)
    plsc.subcore_barrier()

    @pl.when(sid < NS - 1)
    def _():
      pltpu.sync_copy(acc_sh.at[pl.ds(base_row, STRIPE)],
                      out_hbm.at[cid, pl.ds(base_row, STRIPE)])

    @pl.when(sid == NS - 1)
    def _():
      last = (NS - 1) * STRIPE
      pltpu.sync_copy(acc_sh.at[pl.ds(last, N - last)],
                      out_hbm.at[cid, pl.ds(last, N - last)])

  return k(input_, src, dst, w)


BLK = 1000


def _tc_body(x_ref, p_ref, w_ref, o_ref):
  w = w_ref[...]
  o_ref[:, :D] = jnp.dot(x_ref[...], w, preferred_element_type=jnp.float32)
  nb = p_ref[0] + p_ref[1]
  o_ref[:, D:] = jnp.dot(nb, w, preferred_element_type=jnp.float32)


def _tc_transform(input_, partials, W):
  return pl.pallas_call(
      _tc_body,
      grid=(N // BLK,),
      in_specs=[
          pl.BlockSpec((BLK, D), lambda i: (i, 0)),
          pl.BlockSpec((NC, BLK, D), lambda i: (0, i, 0)),
          pl.BlockSpec((D, D), lambda i: (0, 0)),
      ],
      out_specs=pl.BlockSpec((BLK, 2 * D), lambda i: (i, 0)),
      out_shape=jax.ShapeDtypeStruct((N, 2 * D), jnp.float32),
  )(input_, partials, W)


@jax.jit
def kernel(input_, edge_index, edge_weight, W):
  src = edge_index[1].astype(jnp.int32)
  dst = edge_index[0].astype(jnp.int32)
  partials = _sc_segment_sum(input_, src, dst, edge_weight)
  return _tc_transform(input_, partials, W)
